# Initial kernel scaffold; baseline (speedup 1.0000x reference)
import jax
import jax.numpy as jnp
from jax.experimental import pallas as pl


def kernel(x, consecutive_cluster, cluster_non_consecutive, W1, b1, K1, kb1, W2, b2, K2, kb2):
    raise NotImplementedError("write your pallas kernel here")


# speedup vs baseline: 3.0684x; 3.0684x over previous
import functools
import jax
import jax.numpy as jnp
from jax.experimental import pallas as pl

N = 100000
VOX = 729
D = 9
P = 2000
NB = N // P


def _onehot(idx, width):
    cols = jax.lax.broadcasted_iota(jnp.int32, (idx.shape[0], width), 1)
    return (idx[:, None] == cols).astype(jnp.float32)


def _hist_kernel(x_ref, cc_ref, cnc_ref, hcc_ref, hcnc_ref):
    @pl.when(pl.program_id(0) == 0)
    def _():
        hcc_ref[...] = jnp.zeros_like(hcc_ref)
        hcnc_ref[...] = jnp.zeros_like(hcnc_ref)

    x = x_ref[...]
    cc = cc_ref[0, 0, :]
    cnc = cnc_ref[0, 0, :]
    xb4 = jnp.concatenate([x, jnp.ones((P, 1), jnp.float32)], axis=1)
    oh_cc = _onehot(cc, VOX)
    oh_cnc = _onehot(cnc, VOX)
    hcc_ref[...] += jax.lax.dot_general(
        oh_cc, xb4, (((0,), (0,)), ((), ())),
        preferred_element_type=jnp.float32)
    hcnc_ref[...] += jax.lax.dot_general(
        oh_cnc, jnp.ones((P, 1), jnp.float32), (((0,), (0,)), ((), ())),
        preferred_element_type=jnp.float32)


def _rank_perm(pcnt):
    present = (pcnt > 0.0).astype(jnp.float32)
    rows = jax.lax.broadcasted_iota(jnp.int32, (VOX, VOX), 0)
    cols = jax.lax.broadcasted_iota(jnp.int32, (VOX, VOX), 1)
    lower = (cols < rows).astype(jnp.float32)
    rank = jax.lax.dot_general(lower, present, (((1,), (0,)), ((), ())),
                               preferred_element_type=jnp.float32)
    pm = (rank == cols.astype(jnp.float32)).astype(jnp.float32) * present
    return pm


def _conv_grid(grid_vc, kr_ref, kb):
    cin = grid_vc.shape[1]
    cout = kr_ref.shape[2]
    pad = jnp.zeros((96, cin), jnp.float32)
    padded = jnp.concatenate([pad, grid_vc, pad], axis=0)
    v = jax.lax.broadcasted_iota(jnp.int32, (VOX, 1), 0)
    vz = v // 81
    vy = (v // 9) % 9
    vx = v % 9
    acc = jnp.zeros((VOX, cout), jnp.float32)
    for dz in range(3):
        for dy in range(3):
            for dx in range(3):
                o = (dz * 3 + dy) * 3 + dx
                k = (dz - 1) * 81 + (dy - 1) * 9 + (dx - 1)
                shifted = padded[96 + k: 96 + k + VOX, :]
                okz = jnp.logical_and(vz + (dz - 1) >= 0, vz + (dz - 1) < 9)
                oky = jnp.logical_and(vy + (dy - 1) >= 0, vy + (dy - 1) < 9)
                okx = jnp.logical_and(vx + (dx - 1) >= 0, vx + (dx - 1) < 9)
                m = jnp.logical_and(okz, jnp.logical_and(oky, okx))
                contrib = jnp.where(m, shifted, 0.0)
                acc += jax.lax.dot_general(
                    contrib, kr_ref[o], (((1,), (0,)), ((), ())),
                    preferred_element_type=jnp.float32)
    return acc + kb


def _dense1_kernel(hcc_ref, hcnc_ref, w1_ref, b1_ref, k1r_ref, kb1_ref,
                   g1t_ref, s2a_ref):
    sums1 = hcc_ref[:, 0:3]
    cnt = hcc_ref[:, 3:4]
    pm = _rank_perm(hcnc_ref[...])
    means1 = sums1 / jnp.maximum(cnt, 1.0)
    grid1 = jax.lax.dot_general(pm, means1, (((1,), (0,)), ((), ())),
                                preferred_element_type=jnp.float32)
    g1t_ref[...] = _conv_grid(grid1, k1r_ref, kb1_ref[...])
    s2a_ref[...] = jax.lax.dot_general(
        sums1, w1_ref[...], (((1,), (1,)), ((), ())),
        preferred_element_type=jnp.float32) + cnt * b1_ref[...]


def _passb_kernel(cc_ref, cnc_ref, g1t_ref, s2b_ref):
    @pl.when(pl.program_id(0) == 0)
    def _():
        s2b_ref[...] = jnp.zeros_like(s2b_ref)

    cc = cc_ref[0, 0, :]
    cnc = cnc_ref[0, 0, :]
    oh_cnc = _onehot(cnc, VOX)
    oh_cc = _onehot(cc, VOX)
    t = jax.lax.dot_general(oh_cnc, g1t_ref[...], (((1,), (0,)), ((), ())),
                            preferred_element_type=jnp.float32)
    s2b_ref[...] += jax.lax.dot_general(
        oh_cc, t, (((0,), (0,)), ((), ())),
        preferred_element_type=jnp.float32)


def _dense2_kernel(hcc_ref, hcnc_ref, s2a_ref, s2b_ref, g1t_ref,
                   w1_ref, b1_ref, w2_ref, b2_ref, k2r_ref, kb2_ref,
                   gct_ref, at_ref, bp_ref):
    cnt = hcc_ref[:, 3:4]
    pm = _rank_perm(hcnc_ref[...])
    sums2 = s2a_ref[...] + s2b_ref[...]
    means2 = sums2 / jnp.maximum(cnt, 1.0)
    grid2 = jax.lax.dot_general(pm, means2, (((1,), (0,)), ((), ())),
                                preferred_element_type=jnp.float32)
    g2t = _conv_grid(grid2, k2r_ref, kb2_ref[...])
    gct_ref[...] = g2t + jax.lax.dot_general(
        g1t_ref[...], w2_ref[...], (((1,), (1,)), ((), ())),
        preferred_element_type=jnp.float32)
    at_ref[...] = jax.lax.dot_general(
        w1_ref[...], w2_ref[...], (((0,), (1,)), ((), ())),
        preferred_element_type=jnp.float32)
    bp_ref[...] = jax.lax.dot_general(
        b1_ref[...], w2_ref[...], (((1,), (1,)), ((), ())),
        preferred_element_type=jnp.float32) + b2_ref[...]


def _passc_kernel(x_ref, cnc_ref, gct_ref, at_ref, bp_ref, out_ref):
    cnc = cnc_ref[0, 0, :]
    oh_cnc = _onehot(cnc, VOX)
    outer = jax.lax.dot_general(oh_cnc, gct_ref[...], (((1,), (0,)), ((), ())),
                                preferred_element_type=jnp.float32)
    inner = jax.lax.dot_general(x_ref[...], at_ref[...],
                                (((1,), (0,)), ((), ())),
                                preferred_element_type=jnp.float32)
    out_ref[...] = inner + outer + bp_ref[...]


@jax.jit
def kernel(x, consecutive_cluster, cluster_non_consecutive,
           W1, b1, K1, kb1, W2, b2, K2, kb2):
    cc3 = consecutive_cluster.astype(jnp.int32).reshape(NB, 1, P)
    cnc3 = cluster_non_consecutive.astype(jnp.int32).reshape(NB, 1, P)
    k1r = jnp.transpose(K1, (2, 3, 4, 1, 0)).reshape(27, 3, 64)
    k2r = jnp.transpose(K2, (2, 3, 4, 1, 0)).reshape(27, 64, 64)
    b1r = b1.reshape(1, 64)
    b2r = b2.reshape(1, 64)
    kb1r = kb1.reshape(1, 64)
    kb2r = kb2.reshape(1, 64)

    f32 = jnp.float32
    idx_spec = pl.BlockSpec((1, 1, P), lambda i: (i, 0, 0))
    full = lambda s: pl.BlockSpec(s, lambda i: tuple(0 for _ in s))

    hcc, hcnc = pl.pallas_call(
        _hist_kernel,
        grid=(NB,),
        in_specs=[pl.BlockSpec((P, 3), lambda i: (i, 0)), idx_spec, idx_spec],
        out_specs=[full((VOX, 4)), full((VOX, 1))],
        out_shape=[jax.ShapeDtypeStruct((VOX, 4), f32),
                   jax.ShapeDtypeStruct((VOX, 1), f32)],
    )(x, cc3, cnc3)

    g1t, s2a = pl.pallas_call(
        _dense1_kernel,
        out_shape=[jax.ShapeDtypeStruct((VOX, 64), f32),
                   jax.ShapeDtypeStruct((VOX, 64), f32)],
    )(hcc, hcnc, W1, b1r, k1r, kb1r)

    s2b = pl.pallas_call(
        _passb_kernel,
        grid=(NB,),
        in_specs=[idx_spec, idx_spec, full((VOX, 64))],
        out_specs=full((VOX, 64)),
        out_shape=jax.ShapeDtypeStruct((VOX, 64), f32),
    )(cc3, cnc3, g1t)

    gct, at, bp = pl.pallas_call(
        _dense2_kernel,
        out_shape=[jax.ShapeDtypeStruct((VOX, 64), f32),
                   jax.ShapeDtypeStruct((3, 64), f32),
                   jax.ShapeDtypeStruct((1, 64), f32)],
    )(hcc, hcnc, s2a, s2b, g1t, W1, b1r, W2, b2r, k2r, kb2r)

    out = pl.pallas_call(
        _passc_kernel,
        grid=(NB,),
        in_specs=[pl.BlockSpec((P, 3), lambda i: (i, 0)), idx_spec,
                  full((VOX, 64)), full((3, 64)), full((1, 64))],
        out_specs=pl.BlockSpec((P, 64), lambda i: (i, 0)),
        out_shape=jax.ShapeDtypeStruct((N, 64), f32),
    )(x, cnc3, gct, at, bp)
    return out
